# Initial kernel scaffold; baseline (speedup 1.0000x reference)
import jax
import jax.numpy as jnp
from jax.experimental import pallas as pl


def kernel(user, pos, neg, negs, user_embedding, item_embedding):
    raise NotImplementedError("write your pallas kernel here")


# speedup vs baseline: 1.3173x; 1.3173x over previous
import functools

import jax
import jax.numpy as jnp
from jax import lax
from jax.experimental import pallas as pl
from jax.experimental.pallas import tpu as pltpu
from jax.experimental.pallas import tpu_sc as plsc

N_USER = 100000
N_ITEM = 1000000
EMBED = 32
REGS = 0.01
MARGIN = 1.0
BATCH = 16384
N_NEGS = 20

NC = 2
NS = 16
NW = NC * NS
PER_W = BATCH // NW
CHUNK = 128
NCHUNK = PER_W // CHUNK
GROUPS = CHUNK // 16


def _sc_body(user2d, pos2d, neg2d, negs2d, uemb, iemb,
             pos_out, neg_out, rg_out, negs_out,
             uidx_v, pidx_v, nidx_v, gidx_v,
             urows, prows, nrows, grows,
             opos, oneg, org, onegs, sem):
    wid = lax.axis_index("s") * NC + lax.axis_index("c")

    pltpu.sync_copy(user2d.at[pl.ds(wid * 4, 4)], uidx_v)
    pltpu.sync_copy(pos2d.at[pl.ds(wid * 4, 4)], pidx_v)
    pltpu.sync_copy(neg2d.at[pl.ds(wid * 4, 4)], nidx_v)
    pltpu.sync_copy(negs2d.at[pl.ds(wid * 80, 80)], gidx_v)

    iota = lax.iota(jnp.int32, 16)
    iota20 = iota * N_NEGS

    def chunk_body(c, carry):
        h0 = pltpu.async_copy(uemb.at[uidx_v.at[c]], urows, sem)
        h1 = pltpu.async_copy(iemb.at[pidx_v.at[c]], prows, sem)
        h2 = pltpu.async_copy(iemb.at[nidx_v.at[c]], nrows, sem)
        hs = []
        for t in range(N_NEGS):
            hs.append(pltpu.async_copy(
                iemb.at[gidx_v.at[c * N_NEGS + t]],
                grows.at[pl.ds(t * CHUNK, CHUNK)], sem))
        h0.wait()
        h1.wait()
        h2.wait()
        for h in hs:
            h.wait()

        def group_body(g, carry2):
            rb = iota + g * 16
            rows_g = [iota20 + (g * (16 * N_NEGS) + k) for k in range(N_NEGS)]
            z = jnp.zeros((16,), jnp.float32)

            def jbody(j, acc):
                rpos, rneg, rgs, accs = acc
                cj = jnp.full((16,), j, jnp.int32)
                uj = plsc.load_gather(urows, [rb, cj])
                pj = plsc.load_gather(prows, [rb, cj])
                nj = plsc.load_gather(nrows, [rb, cj])
                dp = uj - pj
                dn = uj - nj
                rpos = rpos + dp * dp
                rneg = rneg + dn * dn
                rgs = rgs + uj * uj + pj * pj + nj * nj
                new_accs = []
                for k in range(N_NEGS):
                    xk = plsc.load_gather(grows, [rows_g[k], cj])
                    dk = uj - xk
                    new_accs.append(accs[k] + dk * dk)
                    rgs = rgs + xk * xk
                return (rpos, rneg, rgs, tuple(new_accs))

            init = (z, z, z, (z,) * N_NEGS)
            rpos, rneg, rgs, accs = lax.fori_loop(0, EMBED, jbody, init)

            goff = c * CHUNK + g * 16
            opos[pl.ds(goff, 16)] = rpos
            oneg[pl.ds(goff, 16)] = rneg
            org[pl.ds(goff, 16)] = rgs
            for k in range(N_NEGS):
                onegs[pl.ds(k * PER_W + goff, 16)] = accs[k]
            return carry2

        return lax.fori_loop(0, GROUPS, group_body, carry)

    lax.fori_loop(0, NCHUNK, chunk_body, 0)

    base = wid * PER_W
    pltpu.sync_copy(opos, pos_out.at[pl.ds(base, PER_W)])
    pltpu.sync_copy(oneg, neg_out.at[pl.ds(base, PER_W)])
    pltpu.sync_copy(org, rg_out.at[pl.ds(base, PER_W)])
    for k in range(N_NEGS):
        pltpu.sync_copy(onegs.at[pl.ds(k * PER_W, PER_W)],
                        negs_out.at[k, pl.ds(base, PER_W)])


@functools.partial(
    pl.kernel,
    out_type=(
        jax.ShapeDtypeStruct((BATCH,), jnp.float32),
        jax.ShapeDtypeStruct((BATCH,), jnp.float32),
        jax.ShapeDtypeStruct((BATCH,), jnp.float32),
        jax.ShapeDtypeStruct((N_NEGS, BATCH), jnp.float32),
    ),
    mesh=plsc.VectorSubcoreMesh(core_axis_name="c", subcore_axis_name="s",
                                num_cores=NC, num_subcores=NS),
    compiler_params=pltpu.CompilerParams(
        use_tc_tiling_on_sc=False, needs_layout_passes=False),
    scratch_types=[
        pltpu.VMEM((4, 128), jnp.int32),
        pltpu.VMEM((4, 128), jnp.int32),
        pltpu.VMEM((4, 128), jnp.int32),
        pltpu.VMEM((80, 128), jnp.int32),
        pltpu.VMEM((CHUNK, EMBED), jnp.float32),
        pltpu.VMEM((CHUNK, EMBED), jnp.float32),
        pltpu.VMEM((CHUNK, EMBED), jnp.float32),
        pltpu.VMEM((CHUNK * N_NEGS, EMBED), jnp.float32),
        pltpu.VMEM((PER_W,), jnp.float32),
        pltpu.VMEM((PER_W,), jnp.float32),
        pltpu.VMEM((PER_W,), jnp.float32),
        pltpu.VMEM((PER_W * N_NEGS,), jnp.float32),
        pltpu.SemaphoreType.DMA,
    ],
)
def _sc_kernel(user2d, pos2d, neg2d, negs2d, uemb, iemb,
               pos_out, neg_out, rg_out, negs_out,
               *scratch):
    _sc_body(user2d, pos2d, neg2d, negs2d, uemb, iemb,
             pos_out, neg_out, rg_out, negs_out, *scratch)


def _tc_body(pos_ref, neg_ref, rg_ref, negs_ref, h_ref, r_ref):
    pos_d = jnp.sqrt(pos_ref[...] + 1e-12)
    neg_d = jnp.sqrt(neg_ref[...] + 1e-12)
    s = jnp.sqrt(negs_ref[0] + 1e-12)
    for k in range(1, N_NEGS):
        s = s + jnp.sqrt(negs_ref[k] + 1e-12)
    rank = (pos_d + MARGIN - s * (1.0 / N_NEGS)) * N_USER
    hinge = jnp.sum(jnp.log(rank + 1.0)
                    * jnp.maximum(MARGIN + pos_d - neg_d, 0.0))
    h_ref[...] = jnp.reshape(hinge, (1, 1))
    r_ref[...] = jnp.reshape(REGS * 0.5 * jnp.sum(rg_ref[...]), (1, 1))


_tc_kernel = pl.pallas_call(
    _tc_body,
    out_shape=(
        jax.ShapeDtypeStruct((1, 1), jnp.float32),
        jax.ShapeDtypeStruct((1, 1), jnp.float32),
    ),
)


def kernel(user, pos, neg, negs, user_embedding, item_embedding):
    user2d = user.astype(jnp.int32).reshape(BATCH // 128, 128)
    pos2d = pos.astype(jnp.int32).reshape(BATCH // 128, 128)
    neg2d = neg.astype(jnp.int32).reshape(BATCH // 128, 128)
    negs2d = negs.astype(jnp.int32).reshape(BATCH * N_NEGS // 128, 128)
    pos_ss, neg_ss, rg, negs_ss = _sc_kernel(
        user2d, pos2d, neg2d, negs2d, user_embedding, item_embedding)
    h, r = _tc_kernel(
        pos_ss.reshape(128, 128),
        neg_ss.reshape(128, 128),
        rg.reshape(128, 128),
        negs_ss.reshape(N_NEGS, 128, 128),
    )
    return (h[0, 0], r[0, 0])
